# baseline (speedup 1.0000x reference)
import functools

import jax
import jax.numpy as jnp
import numpy as np
from jax import lax
from jax.experimental import pallas as pl
from jax.experimental.pallas import tpu as pltpu
from jax.experimental.pallas import tpu_sc as plsc

VOCAB = 1000000
DIM = 64
MAX_LEN = 256
B = 4096
S = 200

NUM_CORES = 2
NUM_SUBCORES = 16
NW = NUM_CORES * NUM_SUBCORES
BW = B // NW
NJ = BW // 16


def _sinusoidal_pe(max_len, dim):
    pos = np.arange(max_len, dtype=np.float32)[:, None]
    i = np.arange(0, dim, 2, dtype=np.float32)[None, :]
    angle = pos / np.power(10000.0, i / dim)
    pe = np.zeros((max_len, dim), dtype=np.float32)
    pe[:, 0::2] = np.sin(angle)
    pe[:, 1::2] = np.cos(angle)
    return pe


_PE = _sinusoidal_pe(MAX_LEN, DIM)[:S, :]


@functools.partial(
    pl.kernel,
    mesh=plsc.VectorSubcoreMesh(core_axis_name="c", subcore_axis_name="s"),
    out_type=jax.ShapeDtypeStruct((S, DIM, B), jnp.float32),
    compiler_params=pltpu.CompilerParams(
        use_tc_tiling_on_sc=True, needs_layout_passes=False
    ),
    scratch_types=[
        pltpu.VMEM((S, BW), jnp.int32),
        pltpu.VMEM((2, BW), jnp.int32),
        pltpu.VMEM((2, BW), jnp.int32),
        pltpu.VMEM((BW, 128), jnp.float32),
        pltpu.VMEM((BW, 128), jnp.float32),
        pltpu.VMEM((DIM, BW), jnp.float32),
        pltpu.VMEM((DIM, BW), jnp.float32),
        pltpu.VMEM((S, DIM), jnp.float32),
        pltpu.VMEM((DIM, 16), jnp.float32),
        pltpu.SemaphoreType.DMA,
        pltpu.SemaphoreType.DMA,
        pltpu.SemaphoreType.DMA,
        pltpu.SemaphoreType.DMA,
    ],
)
def _emb(idx_hbm, t2_hbm, pe_hbm, out_hbm,
         idx_v, pidx_v, cb_v, g0, g1, o0, o1, pe_v, psplat_v,
         gsem0, gsem1, osem0, osem1):
    wid = lax.axis_index("s") * NUM_CORES + lax.axis_index("c")
    g = (g0, g1)
    o = (o0, o1)
    gsem = (gsem0, gsem1)
    osem = (osem0, osem1)

    pltpu.sync_copy(pe_hbm, pe_v)
    pltpu.sync_copy(idx_hbm.at[:, pl.ds(wid * BW, BW)], idx_v)

    def prep(s, nb):
        for j in range(NJ):
            sl = pl.ds(j * 16, 16)
            t = idx_v[s, sl]
            pidx_v[nb, sl] = lax.shift_right_logical(t, 1)
            cb_v[nb, sl] = lax.shift_left(jnp.bitwise_and(t, 1), 6)

    def issue_gather(nb):
        pltpu.async_copy(t2_hbm.at[pidx_v.at[nb]], g[nb], gsem[nb])

    def wait_gather(b):
        pltpu.make_async_copy(t2_hbm.at[pl.ds(0, BW)], g[b], gsem[b]).wait()

    def issue_out(s, b):
        pltpu.async_copy(
            o[b], out_hbm.at[s, :, pl.ds(wid * BW, BW)], osem[b]
        )

    def wait_out(b):
        pltpu.make_async_copy(
            o[b], out_hbm.at[0, :, pl.ds(0, BW)], osem[b]
        ).wait()

    def transpose_add(s, b):
        for c in range(DIM // 16):
            pe_c = pe_v[s, pl.ds(c * 16, 16)]
            for dd in range(16):
                psplat_v[c * 16 + dd, :] = jnp.full((16,), pe_c[dd], jnp.float32)

        rowj = []
        cbj = []
        for j in range(NJ):
            rowj.append(lax.iota(jnp.int32, 16) + j * 16)
            cbj.append(cb_v[b, pl.ds(j * 16, 16)])

        def dbody(d, _):
            pe_sp = psplat_v[d, :]
            for j in range(NJ):
                val = plsc.load_gather(g[b], [rowj[j], cbj[j] + d])
                o[b][d, pl.ds(j * 16, 16)] = val + pe_sp
            return 0

        lax.fori_loop(0, DIM, dbody, 0)

    prep(0, 0)
    issue_gather(0)

    def chunk_body(i2, _):
        for b in range(2):
            s = i2 * 2 + b

            @pl.when(s < S - 1)
            def _():
                prep(s + 1, 1 - b)
                issue_gather(1 - b)

            wait_gather(b)

            @pl.when(s >= 2)
            def _():
                wait_out(b)

            transpose_add(s, b)
            issue_out(s, b)
        return 0

    lax.fori_loop(0, S // 2, chunk_body, 0)
    wait_out(0)
    wait_out(1)


def kernel(input, tok_table):
    idx_t = input.T.astype(jnp.int32)
    t2 = tok_table.reshape(VOCAB // 2, 2 * DIM)
    out_t = _emb(idx_t, t2, jnp.asarray(_PE))
    return out_t.transpose(2, 0, 1)


# speedup vs baseline: 1.0083x; 1.0083x over previous
import functools

import jax
import jax.numpy as jnp
import numpy as np
from jax import lax
from jax.experimental import pallas as pl
from jax.experimental.pallas import tpu as pltpu
from jax.experimental.pallas import tpu_sc as plsc

VOCAB = 1000000
DIM = 64
MAX_LEN = 256
B = 4096
S = 200

NUM_CORES = 2
NUM_SUBCORES = 16
NW = NUM_CORES * NUM_SUBCORES
BW = B // NW
NJ = BW // 16


def _sinusoidal_pe(max_len, dim):
    pos = np.arange(max_len, dtype=np.float32)[:, None]
    i = np.arange(0, dim, 2, dtype=np.float32)[None, :]
    angle = pos / np.power(10000.0, i / dim)
    pe = np.zeros((max_len, dim), dtype=np.float32)
    pe[:, 0::2] = np.sin(angle)
    pe[:, 1::2] = np.cos(angle)
    return pe


_PE = _sinusoidal_pe(MAX_LEN, DIM)[:S, :]
_PE_SPLAT = np.repeat(_PE, 16, axis=1).reshape(S, DIM * 16)


@functools.partial(
    pl.kernel,
    mesh=plsc.VectorSubcoreMesh(core_axis_name="c", subcore_axis_name="s"),
    out_type=jax.ShapeDtypeStruct((S, DIM, B), jnp.float32),
    compiler_params=pltpu.CompilerParams(
        use_tc_tiling_on_sc=True, needs_layout_passes=False
    ),
    scratch_types=[
        pltpu.VMEM((S, BW), jnp.int32),
        pltpu.VMEM((2, BW), jnp.int32),
        pltpu.VMEM((2, BW), jnp.int32),
        pltpu.VMEM((BW, 128), jnp.float32),
        pltpu.VMEM((BW, 128), jnp.float32),
        pltpu.VMEM((DIM, BW), jnp.float32),
        pltpu.VMEM((DIM, BW), jnp.float32),
        pltpu.VMEM((2, DIM * 16), jnp.float32),
        pltpu.SemaphoreType.DMA,
        pltpu.SemaphoreType.DMA,
        pltpu.SemaphoreType.DMA,
        pltpu.SemaphoreType.DMA,
    ],
)
def _emb(idx_hbm, t2_hbm, pe_hbm, out_hbm,
         idx_v, pidx_v, cb_v, g0, g1, o0, o1, psplat_v,
         gsem0, gsem1, osem0, osem1):
    wid = lax.axis_index("s") * NUM_CORES + lax.axis_index("c")
    g = (g0, g1)
    o = (o0, o1)
    gsem = (gsem0, gsem1)
    osem = (osem0, osem1)

    pltpu.sync_copy(idx_hbm.at[:, pl.ds(wid * BW, BW)], idx_v)

    def prep(s, nb):
        for j in range(NJ):
            sl = pl.ds(j * 16, 16)
            t = idx_v[s, sl]
            pidx_v[nb, sl] = lax.shift_right_logical(t, 1)
            cb_v[nb, sl] = lax.shift_left(jnp.bitwise_and(t, 1), 6)

    def issue_gather(s, nb):
        pltpu.async_copy(t2_hbm.at[pidx_v.at[nb]], g[nb], gsem[nb])
        pltpu.async_copy(pe_hbm.at[s], psplat_v.at[nb], gsem[nb])

    def wait_gather(b):
        pltpu.make_async_copy(t2_hbm.at[pl.ds(0, BW)], g[b], gsem[b]).wait()
        pltpu.make_async_copy(
            pe_hbm.at[0], psplat_v.at[b], gsem[b]
        ).wait()

    def issue_out(s, b):
        pltpu.async_copy(
            o[b], out_hbm.at[s, :, pl.ds(wid * BW, BW)], osem[b]
        )

    def wait_out(b):
        pltpu.make_async_copy(
            o[b], out_hbm.at[0, :, pl.ds(0, BW)], osem[b]
        ).wait()

    def transpose_add(b):
        rowj = []
        cbj = []
        for j in range(NJ):
            rowj.append(lax.iota(jnp.int32, 16) + j * 16)
            cbj.append(cb_v[b, pl.ds(j * 16, 16)])

        def dbody(i, _):
            for dd in range(4):
                d = i * 4 + dd
                pe_sp = psplat_v[b, pl.ds(d * 16, 16)]
                for j in range(NJ):
                    val = plsc.load_gather(g[b], [rowj[j], cbj[j] + d])
                    o[b][d, pl.ds(j * 16, 16)] = val + pe_sp
            return 0

        lax.fori_loop(0, DIM // 4, dbody, 0)

    prep(0, 0)
    issue_gather(0, 0)

    def chunk_body(i2, _):
        for b in range(2):
            s = i2 * 2 + b

            @pl.when(s < S - 1)
            def _():
                prep(s + 1, 1 - b)
                issue_gather(s + 1, 1 - b)

            wait_gather(b)

            @pl.when(s >= 2)
            def _():
                wait_out(b)

            transpose_add(b)
            issue_out(s, b)
        return 0

    lax.fori_loop(0, S // 2, chunk_body, 0)
    wait_out(0)
    wait_out(1)


def kernel(input, tok_table):
    idx_t = input.T.astype(jnp.int32)
    t2 = tok_table.reshape(VOCAB // 2, 2 * DIM)
    out_t = _emb(idx_t, t2, jnp.asarray(_PE_SPLAT))
    return out_t.transpose(2, 0, 1)


# speedup vs baseline: 1.5471x; 1.5343x over previous
import functools

import jax
import jax.numpy as jnp
import numpy as np
from jax import lax
from jax.experimental import pallas as pl
from jax.experimental.pallas import tpu as pltpu
from jax.experimental.pallas import tpu_sc as plsc

VOCAB = 1000000
DIM = 64
MAX_LEN = 256
B = 4096
S = 200

NUM_CORES = 2
NUM_SUBCORES = 16
NW = NUM_CORES * NUM_SUBCORES
BW = B // NW
NJ = BW // 16


def _sinusoidal_pe(max_len, dim):
    pos = np.arange(max_len, dtype=np.float32)[:, None]
    i = np.arange(0, dim, 2, dtype=np.float32)[None, :]
    angle = pos / np.power(10000.0, i / dim)
    pe = np.zeros((max_len, dim), dtype=np.float32)
    pe[:, 0::2] = np.sin(angle)
    pe[:, 1::2] = np.cos(angle)
    return pe


_PE = _sinusoidal_pe(MAX_LEN, DIM)[:S, :]
_PE_SPLAT = np.repeat(_PE, 16, axis=1).reshape(S, DIM * 16)


@functools.partial(
    pl.kernel,
    mesh=plsc.VectorSubcoreMesh(core_axis_name="c", subcore_axis_name="s"),
    out_type=jax.ShapeDtypeStruct((S, DIM, B), jnp.float32),
    compiler_params=pltpu.CompilerParams(
        use_tc_tiling_on_sc=True, needs_layout_passes=False
    ),
    scratch_types=[
        pltpu.VMEM((S, BW), jnp.int32),
        pltpu.VMEM((2, BW), jnp.int32),
        pltpu.VMEM((2, BW), jnp.int32),
        pltpu.VMEM((BW, 128), jnp.float32),
        pltpu.VMEM((BW, 128), jnp.float32),
        pltpu.VMEM((DIM, BW), jnp.float32),
        pltpu.VMEM((DIM, BW), jnp.float32),
        pltpu.VMEM((2, DIM * 16), jnp.float32),
        pltpu.SemaphoreType.DMA,
        pltpu.SemaphoreType.DMA,
        pltpu.SemaphoreType.DMA,
        pltpu.SemaphoreType.DMA,
    ],
)
def _emb(idx_hbm, t2_hbm, pe_hbm, out_hbm,
         idx_v, pidx_v, cb_v, g0, g1, o0, o1, psplat_v,
         gsem0, gsem1, osem0, osem1):
    wid = lax.axis_index("s") * NUM_CORES + lax.axis_index("c")
    g = (g0, g1)
    o = (o0, o1)
    gsem = (gsem0, gsem1)
    osem = (osem0, osem1)

    pltpu.sync_copy(idx_hbm.at[:, pl.ds(wid * BW, BW)], idx_v)

    def prep(s, nb):
        for j in range(NJ):
            sl = pl.ds(j * 16, 16)
            t = idx_v[s, sl]
            pidx_v[nb, sl] = lax.shift_right_logical(t, 1)
            cb_v[nb, sl] = lax.shift_left(jnp.bitwise_and(t, 1), 6)

    def issue_gather(s, nb):
        pltpu.async_copy(t2_hbm.at[pidx_v.at[nb]], g[nb], gsem[nb])
        pltpu.async_copy(pe_hbm.at[s], psplat_v.at[nb], gsem[nb])

    def wait_gather(b):
        pltpu.make_async_copy(t2_hbm.at[pl.ds(0, BW)], g[b], gsem[b]).wait()
        pltpu.make_async_copy(
            pe_hbm.at[0], psplat_v.at[b], gsem[b]
        ).wait()

    def issue_out(s, b):
        pltpu.async_copy(
            o[b], out_hbm.at[s, :, pl.ds(wid * BW, BW)], osem[b]
        )

    def wait_out(b):
        pltpu.make_async_copy(
            o[b], out_hbm.at[0, :, pl.ds(0, BW)], osem[b]
        ).wait()

    def transpose_add(b):
        rowj = []
        cbj = []
        for j in range(NJ):
            rowj.append(lax.iota(jnp.int32, 16) + j * 16)
            cbj.append(cb_v[b, pl.ds(j * 16, 16)])

        @plsc.parallel_loop(0, DIM, step=1, unroll=4)
        def dbody(d):
            pe_sp = psplat_v[b, pl.ds(d * 16, 16)]
            for j in range(NJ):
                val = plsc.load_gather(g[b], [rowj[j], cbj[j] + d])
                o[b][d, pl.ds(j * 16, 16)] = val + pe_sp

    prep(0, 0)
    issue_gather(0, 0)

    def chunk_body(i2, _):
        for b in range(2):
            s = i2 * 2 + b

            @pl.when(s < S - 1)
            def _():
                prep(s + 1, 1 - b)
                issue_gather(s + 1, 1 - b)

            wait_gather(b)

            @pl.when(s >= 2)
            def _():
                wait_out(b)

            transpose_add(b)
            issue_out(s, b)
        return 0

    lax.fori_loop(0, S // 2, chunk_body, 0)
    wait_out(0)
    wait_out(1)


def kernel(input, tok_table):
    idx_t = input.T.astype(jnp.int32)
    t2 = tok_table.reshape(VOCAB // 2, 2 * DIM)
    out_t = _emb(idx_t, t2, jnp.asarray(_PE_SPLAT))
    return out_t.transpose(2, 0, 1)
